# baseline (speedup 1.0000x reference)
import jax
import jax.numpy as jnp
from jax import lax
from jax.experimental import pallas as pl
from jax.experimental.pallas import tpu as pltpu
from jax.experimental.pallas import tpu_sc as plsc

EMBED_DIM = 64
NUM_CORES = 2
NUM_SUBCORES = 16
NW = NUM_CORES * NUM_SUBCORES
CHUNK = 128
K = 8
GROUP = CHUNK * K


def _gather_body(idx_hbm, table_hbm, out_hbm, idx_v, rows_v, gsem):
    wid = lax.axis_index("s") * NUM_CORES + lax.axis_index("c")
    nchunk = idx_v.shape[0]
    ngroup = nchunk // K
    rows_per_w = nchunk * CHUNK
    base = wid * rows_per_w

    pltpu.sync_copy(idx_hbm.at[wid], idx_v)

    def group(g, carry):
        row0 = g * GROUP
        for j in range(K):
            pltpu.async_copy(
                table_hbm.at[idx_v.at[g * K + j]],
                rows_v.at[pl.ds(j * CHUNK, CHUNK)],
                gsem)
        for j in range(K):
            pltpu.make_async_copy(
                table_hbm.at[idx_v.at[g * K + j]],
                rows_v.at[pl.ds(j * CHUNK, CHUNK)],
                gsem).wait()
        pltpu.sync_copy(rows_v, out_hbm.at[pl.ds(base + row0, GROUP)])
        return carry

    lax.fori_loop(0, ngroup, group, 0)


def kernel(input_x, table):
    batch, hist = input_x.shape
    n = batch * hist
    nchunk = n // (NW * CHUNK)
    idx3 = input_x.astype(jnp.int32).reshape(NW, nchunk, CHUNK)
    mesh = plsc.VectorSubcoreMesh(core_axis_name="c", subcore_axis_name="s")
    out = pl.kernel(
        _gather_body,
        out_type=jax.ShapeDtypeStruct((n, EMBED_DIM), jnp.float32),
        mesh=mesh,
        compiler_params=pltpu.CompilerParams(use_tc_tiling_on_sc=False),
        scratch_types=[
            pltpu.VMEM((nchunk, CHUNK), jnp.int32),
            pltpu.VMEM((GROUP, EMBED_DIM), jnp.float32),
            pltpu.SemaphoreType.DMA,
        ],
    )(idx3, table)
    return out.reshape(batch, hist, EMBED_DIM)


# speedup vs baseline: 1.0100x; 1.0100x over previous
import jax
import jax.numpy as jnp
from jax import lax
from jax.experimental import pallas as pl
from jax.experimental.pallas import tpu as pltpu
from jax.experimental.pallas import tpu_sc as plsc

EMBED_DIM = 64
NUM_CORES = 2
NUM_SUBCORES = 16
NW = NUM_CORES * NUM_SUBCORES
CHUNK = 128
K = 2
GROUP = CHUNK * K
NBUF = 4


def _gather_body(idx_hbm, table_hbm, out_hbm, idx_v, rows_v, gsem, osem):
    wid = lax.axis_index("s") * NUM_CORES + lax.axis_index("c")
    nchunk = idx_v.shape[0]
    ngroup = nchunk // K
    rows_per_w = nchunk * CHUNK
    base = wid * rows_per_w

    pltpu.sync_copy(idx_hbm.at[wid], idx_v)

    def fire(g, b):
        for j in range(K):
            pltpu.async_copy(
                table_hbm.at[idx_v.at[g * K + j]],
                rows_v.at[b].at[pl.ds(j * CHUNK, CHUNK)],
                gsem.at[b])

    def drain(g, b):
        for j in range(K):
            pltpu.make_async_copy(
                table_hbm.at[idx_v.at[g * K + j]],
                rows_v.at[b].at[pl.ds(j * CHUNK, CHUNK)],
                gsem.at[b]).wait()

    def out_copy(g, b):
        return pltpu.make_async_copy(
            rows_v.at[b], out_hbm.at[pl.ds(base + g * GROUP, GROUP)],
            osem.at[b])

    for b in range(NBUF):
        fire(b, b)

    def step(p, carry):
        for b in range(NBUF):
            g = p * NBUF + b
            drain(g, b)
            out_copy(g, b).start()
            nxt = g + NBUF

            @pl.when(nxt < ngroup)
            def _():
                out_copy(g, b).wait()
                fire(nxt, b)
        return carry

    lax.fori_loop(0, ngroup // NBUF, step, 0)

    for b in range(NBUF):
        out_copy(ngroup - NBUF + b, b).wait()


def kernel(input_x, table):
    batch, hist = input_x.shape
    n = batch * hist
    nchunk = n // (NW * CHUNK)
    idx3 = input_x.astype(jnp.int32).reshape(NW, nchunk, CHUNK)
    mesh = plsc.VectorSubcoreMesh(core_axis_name="c", subcore_axis_name="s")
    out = pl.kernel(
        _gather_body,
        out_type=jax.ShapeDtypeStruct((n, EMBED_DIM), jnp.float32),
        mesh=mesh,
        compiler_params=pltpu.CompilerParams(use_tc_tiling_on_sc=False),
        scratch_types=[
            pltpu.VMEM((nchunk, CHUNK), jnp.int32),
            pltpu.VMEM((NBUF, GROUP, EMBED_DIM), jnp.float32),
            pltpu.SemaphoreType.DMA((NBUF,)),
            pltpu.SemaphoreType.DMA((NBUF,)),
        ],
    )(idx3, table)
    return out.reshape(batch, hist, EMBED_DIM)
